# baseline (speedup 1.0000x reference)
import jax
import jax.numpy as jnp
from jax.experimental import pallas as pl
from jax.experimental.pallas import tpu as pltpu


def _row_gather(src_ref, val_ref, x_ref, o_ref):
    i = pl.program_id(0)
    o_ref[...] = x_ref[...] * val_ref[i]


def kernel(x, indices):
    B, C, H, W = x.shape
    K = indices.shape[0]
    HW = H * W
    rows = B * K

    idx = indices.astype(jnp.int32)
    orow = jnp.arange(rows, dtype=jnp.int32)
    b = orow // K
    iv = idx[orow % K]
    valid = (iv < C).astype(x.dtype)
    src = b * C + jnp.where(iv < C, iv, C - 1)

    LN = 128
    SL = HW // LN
    x3 = x.reshape(B * C, SL, LN)
    out3 = pl.pallas_call(
        _row_gather,
        grid_spec=pltpu.PrefetchScalarGridSpec(
            num_scalar_prefetch=2,
            grid=(rows,),
            in_specs=[
                pl.BlockSpec((1, SL, LN), lambda i, src_ref, val_ref: (src_ref[i], 0, 0)),
            ],
            out_specs=pl.BlockSpec((1, SL, LN), lambda i, src_ref, val_ref: (i, 0, 0)),
        ),
        out_shape=jax.ShapeDtypeStruct((rows, SL, LN), x.dtype),
    )(src, valid, x3)
    return out3.reshape(B, K, H, W)


# speedup vs baseline: 3.3335x; 3.3335x over previous
import functools

import jax
import jax.numpy as jnp
from jax import lax
from jax.experimental import pallas as pl
from jax.experimental.pallas import tpu as pltpu
from jax.experimental.pallas import tpu_sc as plsc

_F = 2
_CHUNK = 8
_STRIDE = 16


def _sc_body(rpw, srw, x_hbm, src_hbm, valf_hbm, nv_hbm, zero_hbm, out_hbm,
             src_v, valf_v, nv_v, buf, zbuf, sem):
    wid = lax.axis_index("s") * 2 + lax.axis_index("c")
    base = wid * rpw
    pltpu.sync_copy(src_hbm.at[wid], src_v)
    pltpu.sync_copy(valf_hbm.at[wid], valf_v)
    pltpu.sync_copy(nv_hbm.at[wid], nv_v)
    pltpu.sync_copy(zero_hbm, zbuf)
    nch = rpw // _CHUNK
    for ch in range(nch):
        r0 = base + ch * _CHUNK
        nv = nv_v[pl.ds(ch * _STRIDE, 16)][0]
        idx_ref = src_v.at[pl.ds(ch * _CHUNK, _CHUNK)]

        @pl.when(nv > 0)
        def _gather():
            pltpu.async_copy(x_hbm.at[idx_ref], buf, sem).wait()

            @pl.when(nv < _CHUNK)
            def _mask():
                v16f = valf_v[pl.ds(ch * _STRIDE, 16)]
                for i in range(_CHUNK):
                    bit = v16f[i]

                    def _mul_row(k, _):
                        buf[i, pl.ds(k * 16, 16)] = (
                            buf[i, pl.ds(k * 16, 16)] * bit)
                        return ()

                    lax.fori_loop(0, srw // 16, _mul_row, ())

            pltpu.sync_copy(buf, out_hbm.at[pl.ds(r0, _CHUNK)])

        @pl.when(nv == 0)
        def _zeros():
            pltpu.sync_copy(zbuf, out_hbm.at[pl.ds(r0, _CHUNK)])


def kernel(x, indices):
    B, C, H, W = x.shape
    K = indices.shape[0]
    HW = H * W
    srw = HW // _F
    nrows_out = B * K * _F
    nrows_in = B * C * _F

    info = plsc.get_sparse_core_info()
    nw = info.num_cores * info.num_subcores
    rpw = nrows_out // nw
    nch = rpw // _CHUNK

    idx = indices.astype(jnp.int32)
    srow = jnp.arange(nrows_out, dtype=jnp.int32)
    orow = srow // _F
    sub = srow % _F
    b = orow // K
    iv = idx[orow % K]
    valid = iv < C
    src_short = (b * C + jnp.where(valid, iv, 0)) * _F + sub
    meta_src = src_short.reshape(nw, rpw)
    valid_ch = valid.reshape(nw, nch, _CHUNK)
    nv_chunk = valid_ch.astype(jnp.int32).sum(axis=2)
    meta_nv = (jnp.zeros((nw, nch * _STRIDE), jnp.int32)
               .at[:, ::_STRIDE].set(nv_chunk))
    meta_valf = (jnp.zeros((nw, nch, _STRIDE), jnp.float32)
                 .at[:, :, :_CHUNK].set(valid_ch.astype(jnp.float32))
                 .reshape(nw, nch * _STRIDE))
    zeros_rows = jnp.zeros((_CHUNK, srw), dtype=x.dtype)

    x2 = x.reshape(nrows_in, srw)
    mesh = plsc.VectorSubcoreMesh(core_axis_name="c", subcore_axis_name="s")
    out2 = pl.kernel(
        functools.partial(_sc_body, rpw, srw),
        mesh=mesh,
        out_type=jax.ShapeDtypeStruct((nrows_out, srw), x.dtype),
        scratch_types=[
            pltpu.VMEM((rpw,), jnp.int32),
            pltpu.VMEM((nch * _STRIDE,), jnp.float32),
            pltpu.VMEM((nch * _STRIDE,), jnp.int32),
            pltpu.VMEM((_CHUNK, srw), jnp.float32),
            pltpu.VMEM((_CHUNK, srw), jnp.float32),
            pltpu.SemaphoreType.DMA,
        ],
    )(x2, meta_src, meta_valf, meta_nv, zeros_rows)
    return out2.reshape(B, K, H, W)


# speedup vs baseline: 3.5645x; 1.0693x over previous
import functools

import jax
import jax.numpy as jnp
from jax import lax
from jax.experimental import pallas as pl
from jax.experimental.pallas import tpu as pltpu
from jax.experimental.pallas import tpu_sc as plsc

_F = 2
_CHUNK = 8
_ZROWS = 4
_STRIDE = 16


def _sc_body(rpw, srw, x_hbm, src_hbm, valf_hbm, nv_hbm, zero_hbm, out_hbm,
             src_v, valf_v, nv_v, buf0, buf1, zbuf, gsem, ssem, zsem):
    wid = lax.axis_index("s") * 2 + lax.axis_index("c")
    base = wid * rpw
    pltpu.sync_copy(src_hbm.at[wid], src_v)
    pltpu.sync_copy(valf_hbm.at[wid], valf_v)
    pltpu.sync_copy(nv_hbm.at[wid], nv_v)
    pltpu.sync_copy(zero_hbm, zbuf)
    bufs = (buf0, buf1)
    nch = rpw // _CHUNK

    def nv_of(ch):
        return nv_v[pl.ds(ch * _STRIDE, 16)][0]

    def gather_copy(ch):
        return pltpu.make_async_copy(
            x_hbm.at[src_v.at[pl.ds(ch * _CHUNK, _CHUNK)]],
            bufs[ch % 2], gsem)

    def scatter_copy(ch):
        return pltpu.make_async_copy(
            bufs[ch % 2], out_hbm.at[pl.ds(base + ch * _CHUNK, _CHUNK)],
            ssem)

    def zero_copies(ch):
        r0 = base + ch * _CHUNK
        return (
            pltpu.make_async_copy(
                zbuf, out_hbm.at[pl.ds(r0, _ZROWS)], zsem),
            pltpu.make_async_copy(
                zbuf, out_hbm.at[pl.ds(r0 + _ZROWS, _ZROWS)], zsem),
        )

    for ch in range(nch):
        @pl.when(nv_of(ch) == 0)
        def _fire_zeros():
            za, zb = zero_copies(ch)
            za.start()
            zb.start()

    for ch in range(nch + 1):
        if ch < nch:
            if ch >= 2:
                @pl.when(nv_of(ch - 2) > 0)
                def _free_buf():
                    scatter_copy(ch - 2).wait()

            @pl.when(nv_of(ch) > 0)
            def _start_gather():
                gather_copy(ch).start()

        if ch >= 1:
            pch = ch - 1
            nvp = nv_of(pch)

            @pl.when(nvp > 0)
            def _finish_chunk():
                gather_copy(pch).wait()

                @pl.when(nvp < _CHUNK)
                def _mask():
                    v16f = valf_v[pl.ds(pch * _STRIDE, 16)]
                    pbuf = bufs[pch % 2]
                    for i in range(_CHUNK):
                        bit = v16f[i]

                        def _mul_row(k, _):
                            pbuf[i, pl.ds(k * 16, 16)] = (
                                pbuf[i, pl.ds(k * 16, 16)] * bit)
                            return ()

                        lax.fori_loop(0, srw // 16, _mul_row, ())

                scatter_copy(pch).start()

    for ch in (nch - 2, nch - 1):
        @pl.when(nv_of(ch) > 0)
        def _drain_data():
            scatter_copy(ch).wait()
    for ch in range(nch):
        @pl.when(nv_of(ch) == 0)
        def _drain_zeros():
            za, zb = zero_copies(ch)
            za.wait()
            zb.wait()


def kernel(x, indices):
    B, C, H, W = x.shape
    K = indices.shape[0]
    HW = H * W
    srw = HW // _F
    nrows_out = B * K * _F
    nrows_in = B * C * _F

    info = plsc.get_sparse_core_info()
    nw = info.num_cores * info.num_subcores
    rpw = nrows_out // nw
    nch = rpw // _CHUNK

    idx = indices.astype(jnp.int32)
    srow = jnp.arange(nrows_out, dtype=jnp.int32)
    orow = srow // _F
    sub = srow % _F
    b = orow // K
    iv = idx[orow % K]
    valid = iv < C
    src_short = (b * C + jnp.where(valid, iv, 0)) * _F + sub
    meta_src = src_short.reshape(nw, rpw)
    valid_ch = valid.reshape(nw, nch, _CHUNK)
    nv_chunk = valid_ch.astype(jnp.int32).sum(axis=2)
    meta_nv = (jnp.zeros((nw, nch * _STRIDE), jnp.int32)
               .at[:, ::_STRIDE].set(nv_chunk))
    meta_valf = (jnp.zeros((nw, nch, _STRIDE), jnp.float32)
                 .at[:, :, :_CHUNK].set(valid_ch.astype(jnp.float32))
                 .reshape(nw, nch * _STRIDE))
    zeros_rows = jnp.zeros((_ZROWS, srw), dtype=x.dtype)

    x2 = x.reshape(nrows_in, srw)
    mesh = plsc.VectorSubcoreMesh(core_axis_name="c", subcore_axis_name="s")
    out2 = pl.kernel(
        functools.partial(_sc_body, rpw, srw),
        mesh=mesh,
        out_type=jax.ShapeDtypeStruct((nrows_out, srw), x.dtype),
        scratch_types=[
            pltpu.VMEM((rpw,), jnp.int32),
            pltpu.VMEM((nch * _STRIDE,), jnp.float32),
            pltpu.VMEM((nch * _STRIDE,), jnp.int32),
            pltpu.VMEM((_CHUNK, srw), jnp.float32),
            pltpu.VMEM((_CHUNK, srw), jnp.float32),
            pltpu.VMEM((_ZROWS, srw), jnp.float32),
            pltpu.SemaphoreType.DMA,
            pltpu.SemaphoreType.DMA,
            pltpu.SemaphoreType.DMA,
        ],
    )(x2, meta_src, meta_valf, meta_nv, zeros_rows)
    return out2.reshape(B, K, H, W)


# speedup vs baseline: 9.7962x; 2.7483x over previous
import functools

import jax
import jax.numpy as jnp
from jax import lax
from jax.experimental import pallas as pl
from jax.experimental.pallas import tpu as pltpu
from jax.experimental.pallas import tpu_sc as plsc

_CHUNK = 2
_STRIDE = 16


def _sc_body(ppw, h, w, x_hbm, src_hbm, valf_hbm, nv_hbm, zero_hbm, out_hbm,
             src_v, valf_v, nv_v, buf0, buf1, zbuf, gsem, ssem, zsem):
    wid = lax.axis_index("s") * 2 + lax.axis_index("c")
    base = wid * ppw
    pltpu.sync_copy(src_hbm.at[wid], src_v)
    pltpu.sync_copy(valf_hbm.at[wid], valf_v)
    pltpu.sync_copy(nv_hbm.at[wid], nv_v)
    pltpu.sync_copy(zero_hbm, zbuf)
    bufs = (buf0, buf1)
    nch = ppw // _CHUNK

    def nv_of(ch):
        return nv_v[pl.ds(ch * _STRIDE, 16)][0]

    def gather_copies(ch):
        s16 = src_v[pl.ds(ch * _STRIDE, 16)]
        return tuple(
            pltpu.make_async_copy(
                x_hbm.at[s16[i]], bufs[ch % 2].at[i], gsem)
            for i in range(_CHUNK))

    def scatter_copy(ch):
        return pltpu.make_async_copy(
            bufs[ch % 2], out_hbm.at[pl.ds(base + ch * _CHUNK, _CHUNK)],
            ssem)

    def zero_copy(ch):
        return pltpu.make_async_copy(
            zbuf, out_hbm.at[pl.ds(base + ch * _CHUNK, _CHUNK)], zsem)

    for ch in range(nch):
        @pl.when(nv_of(ch) == 0)
        def _fire_zeros():
            zero_copy(ch).start()

    for ch in range(nch + 1):
        if ch < nch:
            if ch >= 2:
                @pl.when(nv_of(ch - 2) > 0)
                def _free_buf():
                    scatter_copy(ch - 2).wait()

            @pl.when(nv_of(ch) > 0)
            def _start_gather():
                for c in gather_copies(ch):
                    c.start()

        if ch >= 1:
            pch = ch - 1
            nvp = nv_of(pch)

            @pl.when(nvp > 0)
            def _finish_chunk():
                for c in gather_copies(pch):
                    c.wait()

                @pl.when(nvp < _CHUNK)
                def _mask():
                    v16f = valf_v[pl.ds(pch * _STRIDE, 16)]
                    pbuf = bufs[pch % 2]
                    for i in range(_CHUNK):
                        bit = v16f[i]

                        def _mul_row(t, _):
                            r = t // (w // 16)
                            k = t % (w // 16)
                            pbuf[i, r, pl.ds(k * 16, 16)] = (
                                pbuf[i, r, pl.ds(k * 16, 16)] * bit)
                            return ()

                        lax.fori_loop(0, h * (w // 16), _mul_row, ())

                scatter_copy(pch).start()

    for ch in (nch - 2, nch - 1):
        @pl.when(nv_of(ch) > 0)
        def _drain_data():
            scatter_copy(ch).wait()
    for ch in range(nch):
        @pl.when(nv_of(ch) == 0)
        def _drain_zeros():
            zero_copy(ch).wait()


def kernel(x, indices):
    B, C, H, W = x.shape
    K = indices.shape[0]

    info = plsc.get_sparse_core_info()
    nw = info.num_cores * info.num_subcores
    ppw = B * K // nw
    nch = ppw // _CHUNK

    idx = indices.astype(jnp.int32)
    oplane = jnp.arange(B * K, dtype=jnp.int32)
    b = oplane // K
    iv = idx[oplane % K]
    valid = iv < C
    src_plane = b * C + jnp.where(valid, iv, 0)
    meta_src = (jnp.zeros((nw, nch, _STRIDE), jnp.int32)
                .at[:, :, :_CHUNK].set(src_plane.reshape(nw, nch, _CHUNK))
                .reshape(nw, nch * _STRIDE))
    valid_ch = valid.reshape(nw, nch, _CHUNK)
    nv_chunk = valid_ch.astype(jnp.int32).sum(axis=2)
    meta_nv = (jnp.zeros((nw, nch * _STRIDE), jnp.int32)
               .at[:, ::_STRIDE].set(nv_chunk))
    meta_valf = (jnp.zeros((nw, nch, _STRIDE), jnp.float32)
                 .at[:, :, :_CHUNK].set(valid_ch.astype(jnp.float32))
                 .reshape(nw, nch * _STRIDE))
    zeros_planes = jnp.zeros((_CHUNK, H, W), dtype=x.dtype)

    x3 = x.reshape(B * C, H, W)
    mesh = plsc.VectorSubcoreMesh(core_axis_name="c", subcore_axis_name="s")
    out3 = pl.kernel(
        functools.partial(_sc_body, ppw, H, W),
        mesh=mesh,
        out_type=jax.ShapeDtypeStruct((B * K, H, W), x.dtype),
        scratch_types=[
            pltpu.VMEM((nch * _STRIDE,), jnp.int32),
            pltpu.VMEM((nch * _STRIDE,), jnp.float32),
            pltpu.VMEM((nch * _STRIDE,), jnp.int32),
            pltpu.VMEM((_CHUNK, H, W), jnp.float32),
            pltpu.VMEM((_CHUNK, H, W), jnp.float32),
            pltpu.VMEM((_CHUNK, H, W), jnp.float32),
            pltpu.SemaphoreType.DMA,
            pltpu.SemaphoreType.DMA,
            pltpu.SemaphoreType.DMA,
        ],
    )(x3, meta_src, meta_valf, meta_nv, zeros_planes)
    return out3.reshape(B, K, H, W)


# speedup vs baseline: 10.2459x; 1.0459x over previous
import functools

import jax
import jax.numpy as jnp
from jax import lax
from jax.experimental import pallas as pl
from jax.experimental.pallas import tpu as pltpu
from jax.experimental.pallas import tpu_sc as plsc

_CHUNK = 2
_STRIDE = 16


def _sc_body(ppw, h, w, x_hbm, src_hbm, valf_hbm, nv_hbm, zero_hbm, out_hbm,
             src_v, valf_v, nv_v, buf0, buf1, zbuf, gsem, ssem, zsem):
    wid = lax.axis_index("s") * 2 + lax.axis_index("c")
    base = wid * ppw
    pltpu.sync_copy(src_hbm.at[wid], src_v)
    pltpu.sync_copy(valf_hbm.at[wid], valf_v)
    pltpu.sync_copy(nv_hbm.at[wid], nv_v)
    pltpu.sync_copy(zero_hbm, zbuf)
    bufs = (buf0, buf1)
    nch = ppw // _CHUNK

    def nv_of(ch):
        return nv_v[pl.ds(ch * _STRIDE, 16)][0]

    def start_gather(ch, buf):
        s16 = src_v[pl.ds(ch * _STRIDE, 16)]
        for i in range(_CHUNK):
            pltpu.make_async_copy(x_hbm.at[s16[i]], buf.at[i], gsem).start()

    def wait_gather(buf):
        for i in range(_CHUNK):
            pltpu.make_async_copy(x_hbm.at[0], buf.at[i], gsem).wait()

    def scatter_copy(ch, buf):
        return pltpu.make_async_copy(
            buf, out_hbm.at[pl.ds(base + ch * _CHUNK, _CHUNK)], ssem)

    def zero_copy(ch):
        return pltpu.make_async_copy(
            zbuf, out_hbm.at[pl.ds(base + ch * _CHUNK, _CHUNK)], zsem)

    def mask_chunk(ch, buf):
        @pl.when(nv_of(ch) < _CHUNK)
        def _mask():
            v16f = valf_v[pl.ds(ch * _STRIDE, 16)]
            for i in range(_CHUNK):
                bit = v16f[i]

                def _mul_row(t, _):
                    r = t // (w // 16)
                    k = t % (w // 16)
                    buf[i, r, pl.ds(k * 16, 16)] = (
                        buf[i, r, pl.ds(k * 16, 16)] * bit)
                    return ()

                lax.fori_loop(0, h * (w // 16), _mul_row, ())

    def _fire_zeros(ch, carry):
        @pl.when(nv_of(ch) == 0)
        def _z():
            zero_copy(ch).start()
        return carry

    lax.fori_loop(0, nch, _fire_zeros, ())

    def _pipe(it, carry):
        for p in range(2):
            ch = 2 * it + p
            prev = jnp.maximum(ch - 2, 0)

            @pl.when(jnp.logical_and(it > 0, nv_of(prev) > 0))
            def _free_buf():
                scatter_copy(prev, bufs[p]).wait()

            @pl.when(nv_of(ch) > 0)
            def _start():
                start_gather(ch, bufs[p])
        for p in range(2):
            ch = 2 * it + p

            @pl.when(nv_of(ch) > 0)
            def _finish():
                wait_gather(bufs[p])
                mask_chunk(ch, bufs[p])
                scatter_copy(ch, bufs[p]).start()
        return carry

    lax.fori_loop(0, nch // 2, _pipe, ())

    for ch in (nch - 2, nch - 1):
        @pl.when(nv_of(ch) > 0)
        def _drain_data():
            scatter_copy(ch, bufs[ch % 2]).wait()

    def _drain_zeros(ch, carry):
        @pl.when(nv_of(ch) == 0)
        def _z():
            zero_copy(ch).wait()
        return carry

    lax.fori_loop(0, nch, _drain_zeros, ())


def kernel(x, indices):
    B, C, H, W = x.shape
    K = indices.shape[0]

    info = plsc.get_sparse_core_info()
    nw = info.num_cores * info.num_subcores
    ppw = B * K // nw
    nch = ppw // _CHUNK

    idx = indices.astype(jnp.int32)
    oplane = jnp.arange(B * K, dtype=jnp.int32)
    b = oplane // K
    iv = idx[oplane % K]
    valid = iv < C
    src_plane = b * C + jnp.where(valid, iv, 0)
    meta_src = (jnp.zeros((nw, nch, _STRIDE), jnp.int32)
                .at[:, :, :_CHUNK].set(src_plane.reshape(nw, nch, _CHUNK))
                .reshape(nw, nch * _STRIDE))
    valid_ch = valid.reshape(nw, nch, _CHUNK)
    nv_chunk = valid_ch.astype(jnp.int32).sum(axis=2)
    meta_nv = (jnp.zeros((nw, nch * _STRIDE), jnp.int32)
               .at[:, ::_STRIDE].set(nv_chunk))
    meta_valf = (jnp.zeros((nw, nch, _STRIDE), jnp.float32)
                 .at[:, :, :_CHUNK].set(valid_ch.astype(jnp.float32))
                 .reshape(nw, nch * _STRIDE))
    zeros_planes = jnp.zeros((_CHUNK, H, W), dtype=x.dtype)

    x3 = x.reshape(B * C, H, W)
    mesh = plsc.VectorSubcoreMesh(core_axis_name="c", subcore_axis_name="s")
    out3 = pl.kernel(
        functools.partial(_sc_body, ppw, H, W),
        mesh=mesh,
        out_type=jax.ShapeDtypeStruct((B * K, H, W), x.dtype),
        scratch_types=[
            pltpu.VMEM((nch * _STRIDE,), jnp.int32),
            pltpu.VMEM((nch * _STRIDE,), jnp.float32),
            pltpu.VMEM((nch * _STRIDE,), jnp.int32),
            pltpu.VMEM((_CHUNK, H, W), jnp.float32),
            pltpu.VMEM((_CHUNK, H, W), jnp.float32),
            pltpu.VMEM((_CHUNK, H, W), jnp.float32),
            pltpu.SemaphoreType.DMA,
            pltpu.SemaphoreType.DMA,
            pltpu.SemaphoreType.DMA,
        ],
    )(x3, meta_src, meta_valf, meta_nv, zeros_planes)
    return out3.reshape(B, K, H, W)


# speedup vs baseline: 10.6435x; 1.0388x over previous
import functools

import jax
import jax.numpy as jnp
from jax import lax
from jax.experimental import pallas as pl
from jax.experimental.pallas import tpu as pltpu
from jax.experimental.pallas import tpu_sc as plsc

_CHUNK = 2
_STRIDE = 16


def _sc_body(ppw, x_hbm, meta_hbm, zero_hbm, out_hbm,
             meta_v, buf0, buf1, zbuf, gsem, ssem, zsem):
    wid = lax.axis_index("s") * 2 + lax.axis_index("c")
    base = wid * ppw
    pltpu.sync_copy(meta_hbm.at[wid], meta_v)
    pltpu.sync_copy(zero_hbm, zbuf)
    bufs = (buf0, buf1)
    nch = ppw // _CHUNK

    def w16_of(ch):
        return meta_v[pl.ds(ch * _STRIDE, 16)]

    def _fire_zeros(ch, carry):
        w16 = w16_of(ch)
        for i in range(_CHUNK):
            @pl.when(w16[i] < 0)
            def _z():
                pltpu.make_async_copy(
                    zbuf, out_hbm.at[base + ch * _CHUNK + i], zsem).start()
        return carry

    lax.fori_loop(0, nch, _fire_zeros, ())

    def start_gathers(ch, buf):
        w16 = w16_of(ch)
        for i in range(_CHUNK):
            s = w16[i]

            @pl.when(s >= 0)
            def _g():
                pltpu.make_async_copy(x_hbm.at[s], buf.at[i], gsem).start()

    def wait_gathers(ch, buf):
        w16 = w16_of(ch)
        for i in range(_CHUNK):
            @pl.when(w16[i] >= 0)
            def _w():
                pltpu.make_async_copy(x_hbm.at[0], buf.at[i], gsem).wait()

    def scatter(ch, buf, start):
        w16 = w16_of(ch)
        for i in range(_CHUNK):
            @pl.when(w16[i] >= 0)
            def _s():
                c = pltpu.make_async_copy(
                    buf.at[i], out_hbm.at[base + ch * _CHUNK + i], ssem)
                if start:
                    c.start()
                else:
                    c.wait()

    def _pipe(it, carry):
        for p in range(2):
            ch = 2 * it + p
            prev = jnp.maximum(ch - 2, 0)

            @pl.when(it > 0)
            def _free_buf():
                scatter(prev, bufs[p], start=False)

            start_gathers(ch, bufs[p])
        for p in range(2):
            ch = 2 * it + p
            wait_gathers(ch, bufs[p])
            scatter(ch, bufs[p], start=True)
        return carry

    lax.fori_loop(0, nch // 2, _pipe, ())

    for ch in (nch - 2, nch - 1):
        scatter(ch, bufs[ch % 2], start=False)

    def _drain_zeros(ch, carry):
        w16 = w16_of(ch)
        for i in range(_CHUNK):
            @pl.when(w16[i] < 0)
            def _z():
                pltpu.make_async_copy(
                    zbuf, out_hbm.at[base + ch * _CHUNK + i], zsem).wait()
        return carry

    lax.fori_loop(0, nch, _drain_zeros, ())


def kernel(x, indices):
    B, C, H, W = x.shape
    K = indices.shape[0]

    info = plsc.get_sparse_core_info()
    nw = info.num_cores * info.num_subcores
    ppw = B * K // nw
    nch = ppw // _CHUNK

    idx = indices.astype(jnp.int32)
    oplane = jnp.arange(B * K, dtype=jnp.int32)
    b = oplane // K
    iv = idx[oplane % K]
    src = jnp.where(iv < C, b * C + iv, -1)
    meta = (jnp.zeros((nw, nch, _STRIDE), jnp.int32)
            .at[:, :, :_CHUNK].set(src.reshape(nw, nch, _CHUNK))
            .reshape(nw, nch * _STRIDE))
    zero_plane = jnp.zeros((H, W), dtype=x.dtype)

    x3 = x.reshape(B * C, H, W)
    mesh = plsc.VectorSubcoreMesh(core_axis_name="c", subcore_axis_name="s")
    out3 = pl.kernel(
        functools.partial(_sc_body, ppw),
        mesh=mesh,
        out_type=jax.ShapeDtypeStruct((B * K, H, W), x.dtype),
        scratch_types=[
            pltpu.VMEM((nch * _STRIDE,), jnp.int32),
            pltpu.VMEM((_CHUNK, H, W), jnp.float32),
            pltpu.VMEM((_CHUNK, H, W), jnp.float32),
            pltpu.VMEM((H, W), jnp.float32),
            pltpu.SemaphoreType.DMA,
            pltpu.SemaphoreType.DMA,
            pltpu.SemaphoreType.DMA,
        ],
    )(x3, meta, zero_plane)
    return out3.reshape(B, K, H, W)


# speedup vs baseline: 12.6722x; 1.1906x over previous
import functools

import jax
import jax.numpy as jnp
from jax import lax
from jax.experimental import pallas as pl
from jax.experimental.pallas import tpu as pltpu
from jax.experimental.pallas import tpu_sc as plsc

_CHUNK = 2
_STRIDE = 16


def _sc_body(ppw, x_hbm, meta_hbm, zero_hbm, out_hbm,
             meta_v, buf0, buf1, zbuf, gsem, ssem, zsem):
    wid = lax.axis_index("s") * 2 + lax.axis_index("c")
    base = wid * ppw
    pltpu.sync_copy(meta_hbm.at[wid], meta_v)
    pltpu.sync_copy(zero_hbm, zbuf)
    bufs = (buf0, buf1)
    nch = ppw // _CHUNK

    def w16_of(ch):
        return meta_v[pl.ds(ch * _STRIDE, 16)]

    def _fire_zeros(ch, carry):
        w16 = w16_of(ch)
        for i in range(_CHUNK):
            @pl.when(w16[i] < 0)
            def _z():
                pltpu.make_async_copy(
                    zbuf, out_hbm.at[base + ch * _CHUNK + i], zsem).start()
        return carry

    lax.fori_loop(0, nch, _fire_zeros, ())

    def start_gathers(ch, buf):
        w16 = w16_of(ch)
        for i in range(_CHUNK):
            s = w16[i]

            @pl.when(s >= 0)
            def _g():
                pltpu.make_async_copy(x_hbm.at[s], buf.at[i], gsem).start()

    def wait_gathers(ch, buf):
        w16 = w16_of(ch)
        for i in range(_CHUNK):
            @pl.when(w16[i] >= 0)
            def _w():
                pltpu.make_async_copy(x_hbm.at[0], buf.at[i], gsem).wait()

    def scatter(ch, buf, start):
        w16 = w16_of(ch)
        for i in range(_CHUNK):
            @pl.when(w16[i] >= 0)
            def _s():
                c = pltpu.make_async_copy(
                    buf.at[i], out_hbm.at[base + ch * _CHUNK + i], ssem)
                if start:
                    c.start()
                else:
                    c.wait()

    def _pipe(it, carry):
        for p in range(2):
            ch = 2 * it + p
            prev = jnp.maximum(ch - 2, 0)

            @pl.when(it > 0)
            def _free_buf():
                scatter(prev, bufs[p], start=False)

            start_gathers(ch, bufs[p])
        for p in range(2):
            ch = 2 * it + p
            wait_gathers(ch, bufs[p])
            scatter(ch, bufs[p], start=True)
        return carry

    lax.fori_loop(0, nch // 2, _pipe, ())

    for ch in (nch - 2, nch - 1):
        scatter(ch, bufs[ch % 2], start=False)

    def _drain_zeros(ch, carry):
        w16 = w16_of(ch)
        for i in range(_CHUNK):
            @pl.when(w16[i] < 0)
            def _z():
                pltpu.make_async_copy(
                    zbuf, out_hbm.at[base + ch * _CHUNK + i], zsem).wait()
        return carry

    lax.fori_loop(0, nch, _drain_zeros, ())


def kernel(x, indices):
    B, C, H, W = x.shape
    K = indices.shape[0]

    info = plsc.get_sparse_core_info()
    nw = info.num_cores * info.num_subcores
    ppw = B * K // nw
    nch = ppw // _CHUNK

    idx = indices.astype(jnp.int32)
    b2 = jnp.arange(B, dtype=jnp.int32)[:, None]
    iv2 = jnp.broadcast_to(idx[None, :], (B, K))
    src = jnp.where(iv2 < C, b2 * C + iv2, -1).reshape(B * K)
    meta = (jnp.zeros((nw, nch, _STRIDE), jnp.int32)
            .at[:, :, :_CHUNK].set(src.reshape(nw, nch, _CHUNK))
            .reshape(nw, nch * _STRIDE))
    zero_plane = jnp.zeros((H, W), dtype=x.dtype)

    x3 = x.reshape(B * C, H, W)
    mesh = plsc.VectorSubcoreMesh(core_axis_name="c", subcore_axis_name="s")
    out3 = pl.kernel(
        functools.partial(_sc_body, ppw),
        mesh=mesh,
        out_type=jax.ShapeDtypeStruct((B * K, H, W), x.dtype),
        scratch_types=[
            pltpu.VMEM((nch * _STRIDE,), jnp.int32),
            pltpu.VMEM((_CHUNK, H, W), jnp.float32),
            pltpu.VMEM((_CHUNK, H, W), jnp.float32),
            pltpu.VMEM((H, W), jnp.float32),
            pltpu.SemaphoreType.DMA,
            pltpu.SemaphoreType.DMA,
            pltpu.SemaphoreType.DMA,
        ],
    )(x3, meta, zero_plane)
    return out3.reshape(B, K, H, W)


# speedup vs baseline: 12.8418x; 1.0134x over previous
import functools

import jax
import jax.numpy as jnp
from jax import lax
from jax.experimental import pallas as pl
from jax.experimental.pallas import tpu as pltpu
from jax.experimental.pallas import tpu_sc as plsc

_CHUNK = 4
_STRIDE = 16


def _sc_body(ppw, x_hbm, meta_hbm, zero_hbm, out_hbm,
             meta_v, buf0, buf1, zbuf, gsem, ssem, zsem):
    wid = lax.axis_index("s") * 2 + lax.axis_index("c")
    base = wid * ppw
    pltpu.sync_copy(meta_hbm.at[wid], meta_v)
    pltpu.sync_copy(zero_hbm, zbuf)
    bufs = (buf0, buf1)
    nch = ppw // _CHUNK

    def w16_of(ch):
        return meta_v[pl.ds(ch * _STRIDE, 16)]

    def _fire_zeros(ch, carry):
        w16 = w16_of(ch)
        for i in range(_CHUNK):
            @pl.when(w16[i] < 0)
            def _z():
                pltpu.make_async_copy(
                    zbuf, out_hbm.at[base + ch * _CHUNK + i], zsem).start()
        return carry

    lax.fori_loop(0, nch, _fire_zeros, ())

    def start_gathers(ch, buf):
        w16 = w16_of(ch)
        for i in range(_CHUNK):
            s = w16[i]

            @pl.when(s >= 0)
            def _g():
                pltpu.make_async_copy(x_hbm.at[s], buf.at[i], gsem).start()

    def wait_gathers(ch, buf):
        w16 = w16_of(ch)
        for i in range(_CHUNK):
            @pl.when(w16[i] >= 0)
            def _w():
                pltpu.make_async_copy(x_hbm.at[0], buf.at[i], gsem).wait()

    def scatter(ch, buf, start):
        w16 = w16_of(ch)
        for i in range(_CHUNK):
            @pl.when(w16[i] >= 0)
            def _s():
                c = pltpu.make_async_copy(
                    buf.at[i], out_hbm.at[base + ch * _CHUNK + i], ssem)
                if start:
                    c.start()
                else:
                    c.wait()

    def _pipe(it, carry):
        for p in range(2):
            ch = 2 * it + p
            prev = jnp.maximum(ch - 2, 0)

            @pl.when(it > 0)
            def _free_buf():
                scatter(prev, bufs[p], start=False)

            start_gathers(ch, bufs[p])
        for p in range(2):
            ch = 2 * it + p
            wait_gathers(ch, bufs[p])
            scatter(ch, bufs[p], start=True)
        return carry

    lax.fori_loop(0, nch // 2, _pipe, ())

    for ch in (nch - 2, nch - 1):
        scatter(ch, bufs[ch % 2], start=False)

    def _drain_zeros(ch, carry):
        w16 = w16_of(ch)
        for i in range(_CHUNK):
            @pl.when(w16[i] < 0)
            def _z():
                pltpu.make_async_copy(
                    zbuf, out_hbm.at[base + ch * _CHUNK + i], zsem).wait()
        return carry

    lax.fori_loop(0, nch, _drain_zeros, ())


def kernel(x, indices):
    B, C, H, W = x.shape
    K = indices.shape[0]

    info = plsc.get_sparse_core_info()
    nw = info.num_cores * info.num_subcores
    ppw = B * K // nw
    nch = ppw // _CHUNK

    idx = indices.astype(jnp.int32)
    b2 = jnp.arange(B, dtype=jnp.int32)[:, None]
    iv2 = jnp.broadcast_to(idx[None, :], (B, K))
    src = jnp.where(iv2 < C, b2 * C + iv2, -1).reshape(B * K)
    meta = (jnp.zeros((nw, nch, _STRIDE), jnp.int32)
            .at[:, :, :_CHUNK].set(src.reshape(nw, nch, _CHUNK))
            .reshape(nw, nch * _STRIDE))
    zero_plane = jnp.zeros((H, W), dtype=x.dtype)

    x3 = x.reshape(B * C, H, W)
    mesh = plsc.VectorSubcoreMesh(core_axis_name="c", subcore_axis_name="s")
    out3 = pl.kernel(
        functools.partial(_sc_body, ppw),
        mesh=mesh,
        out_type=jax.ShapeDtypeStruct((B * K, H, W), x.dtype),
        scratch_types=[
            pltpu.VMEM((nch * _STRIDE,), jnp.int32),
            pltpu.VMEM((_CHUNK, H, W), jnp.float32),
            pltpu.VMEM((_CHUNK, H, W), jnp.float32),
            pltpu.VMEM((H, W), jnp.float32),
            pltpu.SemaphoreType.DMA,
            pltpu.SemaphoreType.DMA,
            pltpu.SemaphoreType.DMA,
        ],
    )(x3, meta, zero_plane)
    return out3.reshape(B, K, H, W)
